# baseline (speedup 1.0000x reference)
import functools

import jax
import jax.numpy as jnp
from jax import lax
from jax.experimental import pallas as pl
from jax.experimental.pallas import tpu as pltpu
from jax.experimental.pallas import tpu_sc as plsc

N = 10000
E = 320000
D = 128
DE = 16

NCORES = 2
NTILES = 16
EPT = E // NTILES
K = 64
NCHUNK = EPT // K
KT = EPT - NCHUNK * K
NR = 10240
RPT = NR // NTILES
NBUF = 2
NIDX = 4
NGRP = NCHUNK // NIDX
NBLK = RPT // K
XREM = N % K


def _sc_body(x1_hbm, x2_hbm, ei1_hbm, ei2_hbm, ea1_hbm, ea2_hbm, iota_hbm,
             u_out, v_out,
             g_sp, a_sp, deg_sp, src_v, dst_v, xrows_v, earows_v,
             deg_tile, deg_blk, tail_src, tail_dst, idx_v, sem_st, sem_g):
    c = lax.axis_index("c")
    s = lax.axis_index("s")
    rows0 = s * RPT
    ebase = s * EPT

    zero16f = jnp.zeros((16,), jnp.float32)
    ones16f = jnp.ones((16,), jnp.float32)
    lane = lax.iota(jnp.int32, 16)

    def flow(x_hbm, ei_hbm, ea_hbm, vi):
        def stage_copies(i, j):
            base = ebase + i * K
            return (
                pltpu.make_async_copy(ei_hbm.at[0, pl.ds(base, K)],
                                      src_v.at[j], sem_st.at[j]),
                pltpu.make_async_copy(ei_hbm.at[1, pl.ds(base, K)],
                                      dst_v.at[j], sem_st.at[j]),
                pltpu.make_async_copy(ea_hbm.at[pl.ds(base, K)],
                                      earows_v.at[j], sem_st.at[j]),
            )

        def issue_stage(i, j):
            for d in stage_copies(i, j):
                d.start()

        def wait_stage(i, j):
            for d in stage_copies(i, j):
                d.wait()

        def gather_copy(b, j):
            return pltpu.make_async_copy(x_hbm.at[src_v.at[j]],
                                         xrows_v.at[b], sem_g.at[b])

        def hist_update(vals):
            r_idx = jnp.right_shift(vals, 4)
            c_idx = jnp.bitwise_and(vals, 15)
            for l in range(16):
                plsc.addupdate_scatter(deg_tile, [r_idx, c_idx], ones16f,
                                       mask=lane == l)

        def drain_chunk(b, j):
            gather_copy(b, j).wait()
            pltpu.sync_copy(xrows_v.at[b], g_sp.at[dst_v.at[j]], add=True)
            pltpu.sync_copy(earows_v.at[j], a_sp.at[dst_v.at[j]], add=True)
            for t in range(K // 16):
                hist_update(dst_v[j, pl.ds(16 * t, 16)])

        pltpu.sync_copy(iota_hbm, idx_v)

        def zrow(r, _):
            for q in range(D // 16):
                xrows_v[0, r, pl.ds(16 * q, 16)] = zero16f
            earows_v[0, r, :] = zero16f
            return _
        lax.fori_loop(0, K, zrow, 0)
        for t in range(K // 16):
            deg_blk[t, :] = zero16f

        def zdeg(t, _):
            deg_tile[t, :] = zero16f
            return _
        lax.fori_loop(0, NR // 16, zdeg, 0)

        for k in range(NBLK):
            pltpu.sync_copy(xrows_v.at[0], g_sp.at[pl.ds(rows0 + k * K, K)])
            pltpu.sync_copy(earows_v.at[0], a_sp.at[pl.ds(rows0 + k * K, K)])
        pltpu.sync_copy(earows_v.at[0, pl.ds(0, RPT // 16)],
                        deg_sp.at[pl.ds(s * (RPT // 16), RPT // 16)])
        plsc.subcore_barrier()

        for j in range(NBUF):
            issue_stage(j, j)

        def group(g, carry):
            for u in range(NIDX):
                i = g * NIDX + u
                b = u % NBUF

                @pl.when(i >= NBUF)
                def _():
                    drain_chunk(b, (u - NBUF) % NIDX)

                @pl.when(i + NBUF < NCHUNK)
                def _():
                    issue_stage(i + NBUF, (u + NBUF) % NIDX)

                wait_stage(i, u)
                gather_copy(b, u).start()
            return carry

        lax.fori_loop(0, NGRP, group, 0)

        for u in range(NIDX - NBUF, NIDX):
            drain_chunk(u % NBUF, u)

        tbase = ebase + NCHUNK * K
        pltpu.sync_copy(ei_hbm.at[0, pl.ds(tbase, KT)], tail_src)
        pltpu.sync_copy(ei_hbm.at[1, pl.ds(tbase, KT)], tail_dst)
        pltpu.sync_copy(ea_hbm.at[pl.ds(tbase, KT)],
                        earows_v.at[0, pl.ds(0, KT)])
        pltpu.async_copy(x_hbm.at[tail_src], xrows_v.at[0, pl.ds(0, KT)],
                         sem_g.at[0]).wait()
        pltpu.sync_copy(xrows_v.at[0, pl.ds(0, KT)], g_sp.at[tail_dst],
                        add=True)
        pltpu.sync_copy(earows_v.at[0, pl.ds(0, KT)], a_sp.at[tail_dst],
                        add=True)
        for t in range(KT // 16):
            hist_update(tail_dst[pl.ds(16 * t, 16)])

        plsc.subcore_barrier()
        for r in range(NR // 16 // 128):
            pltpu.sync_copy(deg_tile.at[pl.ds(128 * r, 128)],
                            deg_sp.at[idx_v.at[r]], add=True)
        plsc.subcore_barrier()

        for k in range(NBLK):
            blk = rows0 + k * K
            pltpu.sync_copy(g_sp.at[pl.ds(blk, K)], xrows_v.at[0])
            pltpu.sync_copy(a_sp.at[pl.ds(blk, K)], earows_v.at[0])
            pltpu.sync_copy(deg_sp.at[pl.ds(blk // 16, K // 16)], deg_blk)

            for t in range(K // 16):
                deg_blk[t, :] = 1.0 / jnp.maximum(deg_blk[t, :], 1.0)

            def urow(r, _):
                rd = plsc.load_gather(
                    deg_blk,
                    [jnp.full((16,), r // 16, jnp.int32),
                     jnp.full((16,), r % 16, jnp.int32)])
                for q in range(D // 16):
                    sl = pl.ds(16 * q, 16)
                    xrows_v[0, r, sl] = xrows_v[0, r, sl] * rd
                earows_v[0, r, :] = earows_v[0, r, :] * rd
                return _
            lax.fori_loop(0, K, urow, 0)

            for t in range(K // 8):
                pltpu.sync_copy(xrows_v.at[0, pl.ds(8 * t, 8)],
                                u_out.at[vi, blk // 8 + t])
                pltpu.sync_copy(earows_v.at[0, pl.ds(8 * t, 8)],
                                v_out.at[vi, blk // 8 + t])

    @pl.when(c == 0)
    def _():
        flow(x1_hbm, ei1_hbm, ea1_hbm, 0)

    @pl.when(c == 1)
    def _():
        flow(x2_hbm, ei2_hbm, ea2_hbm, 1)


@functools.cache
def _sc_segsum():
    return pl.kernel(
        _sc_body,
        out_type=[
            jax.ShapeDtypeStruct((NCORES, NR // 8, 8, D), jnp.float32),
            jax.ShapeDtypeStruct((NCORES, NR // 8, 8, DE), jnp.float32),
        ],
        mesh=plsc.VectorSubcoreMesh(core_axis_name="c", subcore_axis_name="s",
                                    num_cores=NCORES, num_subcores=NTILES),
        compiler_params=pltpu.CompilerParams(use_tc_tiling_on_sc=False,
                                             needs_layout_passes=False),
        scratch_types=[
            pltpu.VMEM_SHARED((NR, D), jnp.float32),
            pltpu.VMEM_SHARED((NR, DE), jnp.float32),
            pltpu.VMEM_SHARED((NR // 16, 16), jnp.float32),
            pltpu.VMEM((NIDX, K), jnp.int32),
            pltpu.VMEM((NIDX, K), jnp.int32),
            pltpu.VMEM((NBUF, K, D), jnp.float32),
            pltpu.VMEM((NIDX, K, DE), jnp.float32),
            pltpu.VMEM((NR // 16, 16), jnp.float32),
            pltpu.VMEM((K // 16, 16), jnp.float32),
            pltpu.VMEM((KT,), jnp.int32),
            pltpu.VMEM((KT,), jnp.int32),
            pltpu.VMEM((NR // 16 // 128, 128), jnp.int32),
            pltpu.SemaphoreType.DMA((NIDX,)),
            pltpu.SemaphoreType.DMA((NBUF,)),
        ],
    )


R = 2048


def _tc_body(x1_ref, x2_ref, u_ref, v_ref,
             wsn_ref, wse_ref, bs_ref, wtn_ref, wte_ref, bt_ref,
             p1w_ref, p1b_ref, p2w_ref, p2b_ref, out_ref):
    i = pl.program_id(0)

    @pl.when(i == 0)
    def _():
        out_ref[...] = jnp.zeros_like(out_ref)

    wsn = wsn_ref[...]
    wse = wse_ref[...]
    wtn = wtn_ref[...]
    wte = wte_ref[...]

    def view(v):
        x = x1_ref[...] if v == 0 else x2_ref[...]
        u = u_ref[v].reshape(R, D) + x
        vv = v_ref[v]
        hs = jax.nn.relu(jnp.dot(u, wsn, preferred_element_type=jnp.float32)
                         + jnp.dot(vv, wse,
                                   preferred_element_type=jnp.float32
                                   ).reshape(R, D)
                         + bs_ref[...])
        ht = jax.nn.relu(jnp.dot(u, wtn, preferred_element_type=jnp.float32)
                         + jnp.dot(vv, wte,
                                   preferred_element_type=jnp.float32
                                   ).reshape(R, D)
                         + bt_ref[...])
        p = jnp.dot(jax.nn.relu(jnp.dot(hs, p1w_ref[...],
                                        preferred_element_type=jnp.float32)
                                + p1b_ref[...]),
                    p2w_ref[...], preferred_element_type=jnp.float32) + p2b_ref[...]
        return p, ht

    p1, ht1 = view(0)
    p2, ht2 = view(1)

    def cos(a, b):
        na = jnp.sqrt(jnp.sum(a * a, axis=-1, keepdims=True)) + 1e-12
        nb = jnp.sqrt(jnp.sum(b * b, axis=-1, keepdims=True)) + 1e-12
        return jnp.sum(a * b, axis=-1, keepdims=True) / (na * nb)

    lblk = (2.0 - 2.0 * cos(p1, ht2)) + (2.0 - 2.0 * cos(p2, ht1))
    rows = i * R + lax.broadcasted_iota(jnp.int32, (R, 1), 0)
    s_val = jnp.sum(jnp.where(rows < N, lblk, 0.0)) / float(N)
    out_ref[...] = out_ref[...] + s_val


def _tc_call(x1, x2, u, v, wsn, wse2, bs2, wtn, wte2, bt2, p1w, p1b2, p2w, p2b2):
    wspec = pl.BlockSpec((D, D), lambda i: (0, 0))
    bspec = pl.BlockSpec((1, D), lambda i: (0, 0))
    out = pl.pallas_call(
        _tc_body,
        grid=(NR // R,),
        in_specs=[
            pl.BlockSpec((R, D), lambda i: (i, 0)),
            pl.BlockSpec((R, D), lambda i: (i, 0)),
            pl.BlockSpec((NCORES, R // 8, 8, D), lambda i: (0, i, 0, 0)),
            pl.BlockSpec((NCORES, R // 8, 8 * DE), lambda i: (0, i, 0)),
            wspec, pl.BlockSpec((D, 8 * D), lambda i: (0, 0)), bspec,
            wspec, pl.BlockSpec((D, 8 * D), lambda i: (0, 0)), bspec,
            wspec, bspec, wspec, bspec,
        ],
        out_specs=pl.BlockSpec((1, 1), lambda i: (0, 0)),
        out_shape=jax.ShapeDtypeStruct((1, 1), jnp.float32),
    )(x1, x2, u, v, wsn, wse2, bs2, wtn, wte2, bt2, p1w, p1b2, p2w, p2b2)
    return out


@jax.jit
def kernel(x1, x2, edge_index_v1, edge_index_v2, edge_attr1, edge_attr2, batch,
           Ws_node, Ws_edge, bs, Wt_node, Wt_edge, bt, P1_w, P1_b, P2_w, P2_b):
    iota_idx = jnp.arange(NR // 16, dtype=jnp.int32).reshape(-1, 128)
    u, v = _sc_segsum()(x1, x2, edge_index_v1, edge_index_v2,
                        edge_attr1, edge_attr2, iota_idx)
    v = v.reshape(NCORES, NR // 8, 8 * DE)
    eye8 = jnp.eye(8, dtype=jnp.float32)
    out = _tc_call(x1, x2, u, v,
                   Ws_node, jnp.kron(eye8, Ws_edge), bs.reshape(1, D),
                   Wt_node, jnp.kron(eye8, Wt_edge), bt.reshape(1, D),
                   P1_w, P1_b.reshape(1, D), P2_w, P2_b.reshape(1, D))
    return out[0, 0]


# speedup vs baseline: 1.0359x; 1.0359x over previous
import functools

import jax
import jax.numpy as jnp
from jax import lax
from jax.experimental import pallas as pl
from jax.experimental.pallas import tpu as pltpu
from jax.experimental.pallas import tpu_sc as plsc

N = 10000
E = 320000
D = 128
DE = 16

NCORES = 2
NTILES = 16
EPT = E // NTILES
K = 64
NCHUNK = EPT // K
KT = EPT - NCHUNK * K
NR = 10240
RPT = NR // NTILES
NBUF = 2
NIDX = 4
NGRP = NCHUNK // NIDX
NBLK = RPT // K
XREM = N % K


def _sc_body(x1_hbm, x2_hbm, ei1_hbm, ei2_hbm, ea1_hbm, ea2_hbm, iota_hbm,
             u_out, v_out,
             g_sp, a_sp, deg_sp, src_v, dst_v, xrows_v, earows_v,
             deg_tile, deg_blk, tail_src, tail_dst, idx_v, vpack,
             sem_st, sem_g):
    c = lax.axis_index("c")
    s = lax.axis_index("s")
    rows0 = s * RPT
    ebase = s * EPT

    zero16f = jnp.zeros((16,), jnp.float32)
    ones16f = jnp.ones((16,), jnp.float32)
    lane = lax.iota(jnp.int32, 16)

    def flow(x_hbm, ei_hbm, ea_hbm, vi):
        def stage_copies(i, j):
            base = ebase + i * K
            return (
                pltpu.make_async_copy(ei_hbm.at[0, pl.ds(base, K)],
                                      src_v.at[j], sem_st.at[j]),
                pltpu.make_async_copy(ei_hbm.at[1, pl.ds(base, K)],
                                      dst_v.at[j], sem_st.at[j]),
                pltpu.make_async_copy(ea_hbm.at[pl.ds(base, K)],
                                      earows_v.at[j], sem_st.at[j]),
            )

        def issue_stage(i, j):
            for d in stage_copies(i, j):
                d.start()

        def wait_stage(i, j):
            for d in stage_copies(i, j):
                d.wait()

        def gather_copy(b, j):
            return pltpu.make_async_copy(x_hbm.at[src_v.at[j]],
                                         xrows_v.at[b], sem_g.at[b])

        def hist_update(vals):
            r_idx = jnp.right_shift(vals, 4)
            c_idx = jnp.bitwise_and(vals, 15)
            for l in range(16):
                plsc.addupdate_scatter(deg_tile, [r_idx, c_idx], ones16f,
                                       mask=lane == l)

        def drain_chunk(b, j):
            gather_copy(b, j).wait()
            pltpu.sync_copy(xrows_v.at[b], g_sp.at[dst_v.at[j]], add=True)
            pltpu.sync_copy(earows_v.at[j], a_sp.at[dst_v.at[j]], add=True)
            for t in range(K // 16):
                hist_update(dst_v[j, pl.ds(16 * t, 16)])

        pltpu.sync_copy(iota_hbm, idx_v)

        def zrow(r, _):
            for q in range(D // 16):
                xrows_v[0, r, pl.ds(16 * q, 16)] = zero16f
            earows_v[0, r, :] = zero16f
            return _
        lax.fori_loop(0, K, zrow, 0)
        for t in range(K // 16):
            deg_blk[t, :] = zero16f

        def zdeg(t, _):
            deg_tile[t, :] = zero16f
            return _
        lax.fori_loop(0, NR // 16, zdeg, 0)

        for k in range(NBLK):
            pltpu.sync_copy(xrows_v.at[0], g_sp.at[pl.ds(rows0 + k * K, K)])
            pltpu.sync_copy(earows_v.at[0], a_sp.at[pl.ds(rows0 + k * K, K)])
        pltpu.sync_copy(earows_v.at[0, pl.ds(0, RPT // 16)],
                        deg_sp.at[pl.ds(s * (RPT // 16), RPT // 16)])
        plsc.subcore_barrier()

        for j in range(NBUF):
            issue_stage(j, j)

        def group(g, carry):
            for u in range(NIDX):
                i = g * NIDX + u
                b = u % NBUF

                @pl.when(i >= NBUF)
                def _():
                    drain_chunk(b, (u - NBUF) % NIDX)

                @pl.when(i + NBUF < NCHUNK)
                def _():
                    issue_stage(i + NBUF, (u + NBUF) % NIDX)

                wait_stage(i, u)
                gather_copy(b, u).start()
            return carry

        lax.fori_loop(0, NGRP, group, 0)

        for u in range(NIDX - NBUF, NIDX):
            drain_chunk(u % NBUF, u)

        tbase = ebase + NCHUNK * K
        pltpu.sync_copy(ei_hbm.at[0, pl.ds(tbase, KT)], tail_src)
        pltpu.sync_copy(ei_hbm.at[1, pl.ds(tbase, KT)], tail_dst)
        pltpu.sync_copy(ea_hbm.at[pl.ds(tbase, KT)],
                        earows_v.at[0, pl.ds(0, KT)])
        pltpu.async_copy(x_hbm.at[tail_src], xrows_v.at[0, pl.ds(0, KT)],
                         sem_g.at[0]).wait()
        pltpu.sync_copy(xrows_v.at[0, pl.ds(0, KT)], g_sp.at[tail_dst],
                        add=True)
        pltpu.sync_copy(earows_v.at[0, pl.ds(0, KT)], a_sp.at[tail_dst],
                        add=True)
        for t in range(KT // 16):
            hist_update(tail_dst[pl.ds(16 * t, 16)])

        plsc.subcore_barrier()
        for r in range(NR // 16 // 128):
            pltpu.sync_copy(deg_tile.at[pl.ds(128 * r, 128)],
                            deg_sp.at[idx_v.at[r]], add=True)
        plsc.subcore_barrier()

        for k in range(NBLK):
            blk = rows0 + k * K
            pltpu.sync_copy(g_sp.at[pl.ds(blk, K)], xrows_v.at[0])
            pltpu.sync_copy(a_sp.at[pl.ds(blk, K)], earows_v.at[0])
            pltpu.sync_copy(deg_sp.at[pl.ds(blk // 16, K // 16)], deg_blk)

            for t in range(K // 16):
                deg_blk[t, :] = 1.0 / jnp.maximum(deg_blk[t, :], 1.0)

            def urow(r, _):
                rd = plsc.load_gather(
                    deg_blk,
                    [jnp.full((16,), r // 16, jnp.int32),
                     jnp.full((16,), r % 16, jnp.int32)])
                for q in range(D // 16):
                    sl = pl.ds(16 * q, 16)
                    xrows_v[0, r, sl] = xrows_v[0, r, sl] * rd
                vpack[r // 8, pl.ds(16 * (r % 8), 16)] = earows_v[0, r, :] * rd
                return _
            lax.fori_loop(0, K, urow, 0)

            for t in range(K // 8):
                pltpu.sync_copy(xrows_v.at[0, pl.ds(8 * t, 8)],
                                u_out.at[vi, blk // 8 + t])
            pltpu.sync_copy(vpack, v_out.at[vi, pl.ds(blk // 8, K // 8)])

    @pl.when(c == 0)
    def _():
        flow(x1_hbm, ei1_hbm, ea1_hbm, 0)

    @pl.when(c == 1)
    def _():
        flow(x2_hbm, ei2_hbm, ea2_hbm, 1)


@functools.cache
def _sc_segsum():
    return pl.kernel(
        _sc_body,
        out_type=[
            jax.ShapeDtypeStruct((NCORES, NR // 8, 8, D), jnp.float32),
            jax.ShapeDtypeStruct((NCORES, NR // 8, 8 * DE), jnp.float32),
        ],
        mesh=plsc.VectorSubcoreMesh(core_axis_name="c", subcore_axis_name="s",
                                    num_cores=NCORES, num_subcores=NTILES),
        compiler_params=pltpu.CompilerParams(use_tc_tiling_on_sc=False,
                                             needs_layout_passes=False),
        scratch_types=[
            pltpu.VMEM_SHARED((NR, D), jnp.float32),
            pltpu.VMEM_SHARED((NR, DE), jnp.float32),
            pltpu.VMEM_SHARED((NR // 16, 16), jnp.float32),
            pltpu.VMEM((NIDX, K), jnp.int32),
            pltpu.VMEM((NIDX, K), jnp.int32),
            pltpu.VMEM((NBUF, K, D), jnp.float32),
            pltpu.VMEM((NIDX, K, DE), jnp.float32),
            pltpu.VMEM((NR // 16, 16), jnp.float32),
            pltpu.VMEM((K // 16, 16), jnp.float32),
            pltpu.VMEM((KT,), jnp.int32),
            pltpu.VMEM((KT,), jnp.int32),
            pltpu.VMEM((NR // 16 // 128, 128), jnp.int32),
            pltpu.VMEM((K // 8, 8 * DE), jnp.float32),
            pltpu.SemaphoreType.DMA((NIDX,)),
            pltpu.SemaphoreType.DMA((NBUF,)),
        ],
    )


R = 2048


def _tc_body(x1_ref, x2_ref, u_ref, v_ref,
             wsn_ref, wse_ref, bs_ref, wtn_ref, wte_ref, bt_ref,
             p1w_ref, p1b_ref, p2w_ref, p2b_ref, out_ref):
    i = pl.program_id(0)

    @pl.when(i == 0)
    def _():
        out_ref[...] = jnp.zeros_like(out_ref)

    wsn = wsn_ref[...]
    wse = wse_ref[...]
    wtn = wtn_ref[...]
    wte = wte_ref[...]

    def view(v):
        x = x1_ref[...] if v == 0 else x2_ref[...]
        u = u_ref[v].reshape(R, D) + x
        vv = v_ref[v]
        hs = jax.nn.relu(jnp.dot(u, wsn, preferred_element_type=jnp.float32)
                         + jnp.dot(vv, wse,
                                   preferred_element_type=jnp.float32
                                   ).reshape(R, D)
                         + bs_ref[...])
        ht = jax.nn.relu(jnp.dot(u, wtn, preferred_element_type=jnp.float32)
                         + jnp.dot(vv, wte,
                                   preferred_element_type=jnp.float32
                                   ).reshape(R, D)
                         + bt_ref[...])
        p = jnp.dot(jax.nn.relu(jnp.dot(hs, p1w_ref[...],
                                        preferred_element_type=jnp.float32)
                                + p1b_ref[...]),
                    p2w_ref[...], preferred_element_type=jnp.float32) + p2b_ref[...]
        return p, ht

    p1, ht1 = view(0)
    p2, ht2 = view(1)

    def cos(a, b):
        na = jnp.sqrt(jnp.sum(a * a, axis=-1, keepdims=True)) + 1e-12
        nb = jnp.sqrt(jnp.sum(b * b, axis=-1, keepdims=True)) + 1e-12
        return jnp.sum(a * b, axis=-1, keepdims=True) / (na * nb)

    lblk = (2.0 - 2.0 * cos(p1, ht2)) + (2.0 - 2.0 * cos(p2, ht1))
    rows = i * R + lax.broadcasted_iota(jnp.int32, (R, 1), 0)
    s_val = jnp.sum(jnp.where(rows < N, lblk, 0.0)) / float(N)
    out_ref[...] = out_ref[...] + s_val


def _tc_call(x1, x2, u, v, wsn, wse2, bs2, wtn, wte2, bt2, p1w, p1b2, p2w, p2b2):
    wspec = pl.BlockSpec((D, D), lambda i: (0, 0))
    bspec = pl.BlockSpec((1, D), lambda i: (0, 0))
    out = pl.pallas_call(
        _tc_body,
        grid=(NR // R,),
        in_specs=[
            pl.BlockSpec((R, D), lambda i: (i, 0)),
            pl.BlockSpec((R, D), lambda i: (i, 0)),
            pl.BlockSpec((NCORES, R // 8, 8, D), lambda i: (0, i, 0, 0)),
            pl.BlockSpec((NCORES, R // 8, 8 * DE), lambda i: (0, i, 0)),
            wspec, pl.BlockSpec((D, 8 * D), lambda i: (0, 0)), bspec,
            wspec, pl.BlockSpec((D, 8 * D), lambda i: (0, 0)), bspec,
            wspec, bspec, wspec, bspec,
        ],
        out_specs=pl.BlockSpec((1, 1), lambda i: (0, 0)),
        out_shape=jax.ShapeDtypeStruct((1, 1), jnp.float32),
    )(x1, x2, u, v, wsn, wse2, bs2, wtn, wte2, bt2, p1w, p1b2, p2w, p2b2)
    return out


@jax.jit
def kernel(x1, x2, edge_index_v1, edge_index_v2, edge_attr1, edge_attr2, batch,
           Ws_node, Ws_edge, bs, Wt_node, Wt_edge, bt, P1_w, P1_b, P2_w, P2_b):
    iota_idx = jnp.arange(NR // 16, dtype=jnp.int32).reshape(-1, 128)
    u, v = _sc_segsum()(x1, x2, edge_index_v1, edge_index_v2,
                        edge_attr1, edge_attr2, iota_idx)
    eye8 = jnp.eye(8, dtype=jnp.float32)
    out = _tc_call(x1, x2, u, v,
                   Ws_node, jnp.kron(eye8, Ws_edge), bs.reshape(1, D),
                   Wt_node, jnp.kron(eye8, Wt_edge), bt.reshape(1, D),
                   P1_w, P1_b.reshape(1, D), P2_w, P2_b.reshape(1, D))
    return out[0, 0]
